# Initial kernel scaffold; baseline (speedup 1.0000x reference)
import functools

import jax
import jax.numpy as jnp
from jax import lax
from jax.experimental import pallas as pl
from jax.experimental.pallas import tpu as pltpu
from jax.experimental.pallas import tpu_sc as plsc

N, D, G = 10000, 128, 16
NC, NS = 2, 16
NW = NC * NS
CHUNK = N // NW
TAIL = N - NW * CHUNK
BUF = CHUNK + TAIL
NJ = D // 16


def _sc_partial_segsum(x, ids):
    mesh = plsc.VectorSubcoreMesh(core_axis_name="c", subcore_axis_name="s")

    @functools.partial(
        pl.kernel,
        out_type=(
            jax.ShapeDtypeStruct((NW, G, D), jnp.float32),
            jax.ShapeDtypeStruct((NW, G), jnp.float32),
        ),
        mesh=mesh,
        scratch_types=[
            pltpu.VMEM((BUF, D), jnp.float32),
            pltpu.VMEM((BUF,), jnp.int32),
            pltpu.VMEM((G, D), jnp.float32),
            pltpu.VMEM((G,), jnp.float32),
        ],
    )
    def k(x_hbm, ids_hbm, pacc_hbm, pcnt_hbm, x_v, ids_v, acc_v, cnt_v):
        wid = lax.axis_index("s") * NC + lax.axis_index("c")
        base = wid * CHUNK
        pltpu.sync_copy(x_hbm.at[pl.ds(base, CHUNK)], x_v.at[pl.ds(0, CHUNK)])
        pltpu.sync_copy(ids_hbm.at[pl.ds(base, CHUNK)], ids_v.at[pl.ds(0, CHUNK)])
        is_last = wid == NW - 1

        @pl.when(is_last)
        def _():
            pltpu.sync_copy(x_hbm.at[pl.ds(NW * CHUNK, TAIL)],
                            x_v.at[pl.ds(CHUNK, TAIL)])
            pltpu.sync_copy(ids_hbm.at[pl.ds(NW * CHUNK, TAIL)],
                            ids_v.at[pl.ds(CHUNK, TAIL)])

        zero16 = jnp.zeros((16,), jnp.float32)
        for s in range(G):
            for j in range(NJ):
                acc_v[s, pl.ds(16 * j, 16)] = zero16

        nrows = jnp.where(is_last, CHUNK + TAIL, CHUNK)
        lane = lax.iota(jnp.int32, 16)
        prev0 = ids_v[0]
        init = (prev0, jnp.float32(0.0), zero16) + tuple(zero16 for _ in range(NJ))

        def body(r, carry):
            prev, run, cntvec = carry[0], carry[1], carry[2]
            accs = carry[3:]
            sid = ids_v[r]
            flush = sid != prev

            @pl.when(flush)
            def _():
                for j in range(NJ):
                    acc_v[prev, pl.ds(16 * j, 16)] = accs[j]

            new_accs = tuple(
                jnp.where(flush, 0.0, accs[j]) + x_v[r, pl.ds(16 * j, 16)]
                for j in range(NJ)
            )
            cntvec = jnp.where(flush & (lane == prev), run, cntvec)
            run = jnp.where(flush, 0.0, run) + 1.0
            return (sid, run, cntvec) + new_accs

        final = lax.fori_loop(0, nrows, body, init)
        prev, run, cntvec = final[0], final[1], final[2]
        accs = final[3:]
        for j in range(NJ):
            acc_v[prev, pl.ds(16 * j, 16)] = accs[j]
        cnt_v[...] = jnp.where(lane == prev, run, cntvec)

        pltpu.sync_copy(acc_v, pacc_hbm.at[wid])
        pltpu.sync_copy(cnt_v, pcnt_hbm.at[wid])

    return k(x, ids)


def _tc_finalize(pacc, pcnt, Wc, bc):
    def k(pacc_ref, pcnt_ref, wc_ref, bc_ref, o_ref):
        acc = pacc_ref[0]
        for i in range(1, NW):
            acc = acc + pacc_ref[i]
        ones = jnp.ones((NW, 1), jnp.float32)
        cnt = lax.dot_general(pcnt_ref[...], ones,
                              (((0,), (0,)), ((), ())),
                              preferred_element_type=jnp.float32)
        pooled = acc / jnp.maximum(cnt, 1.0)
        out = jnp.dot(pooled, wc_ref[...],
                      preferred_element_type=jnp.float32)
        o_ref[...] = out + bc_ref[...]

    return pl.pallas_call(
        k,
        out_shape=jax.ShapeDtypeStruct((G, Wc.shape[1]), jnp.float32),
    )(pacc, pcnt, Wc, bc)


def kernel(x, edge_index, edge_attr, batch_size, W1, b1, W2, b2, Wc, bc):
    pacc, pcnt = _sc_partial_segsum(x, batch_size)
    return _tc_finalize(pacc, pcnt, Wc, bc.reshape(1, -1))


# speedup vs baseline: 3.2648x; 3.2648x over previous
import functools

import jax
import jax.numpy as jnp
from jax import lax
from jax.experimental import pallas as pl
from jax.experimental.pallas import tpu as pltpu
from jax.experimental.pallas import tpu_sc as plsc

N, D, G = 10000, 128, 16
NC, NS = 1, 16
NW = NC * NS
CHUNK = -(-N // NW // 16) * 16
LASTC = N - (NW - 1) * CHUNK
NGRP = CHUNK // 16
LGRP = LASTC // 16
NJ = D // 16
OPAD = 16


def _sc_pool_classify(xf, ids, wct, bc16):
    mesh = plsc.VectorSubcoreMesh(core_axis_name="c", subcore_axis_name="s",
                                  num_cores=NC, num_subcores=NS)

    @functools.partial(
        pl.kernel,
        out_type=jax.ShapeDtypeStruct((G, OPAD), jnp.float32),
        mesh=mesh,
        scratch_types=[
            pltpu.VMEM((CHUNK * D,), jnp.float32),
            pltpu.VMEM((CHUNK,), jnp.int32),
            pltpu.VMEM((G * D,), jnp.float32),
            pltpu.VMEM((G,), jnp.float32),
            pltpu.VMEM((8 * D,), jnp.float32),
            pltpu.VMEM((16,), jnp.float32),
            pltpu.VMEM((D,), jnp.float32),
            pltpu.VMEM((NW * G,), jnp.float32),
            pltpu.VMEM((OPAD,), jnp.float32),
            pltpu.VMEM_SHARED((NW * G * D,), jnp.float32),
            pltpu.VMEM_SHARED((NW * G,), jnp.float32),
        ],
    )
    def k(x_hbm, ids_hbm, wct_hbm, bc_hbm, out_hbm,
          x_v, ids_v, acc_v, cnt_v, wct_v, bc_v, red_v, rcnt_v, out_v,
          sh_acc, sh_cnt):
        wid = lax.axis_index("s") * NC + lax.axis_index("c")
        base = wid * CHUNK
        is_last = wid == NW - 1

        PIECE = 320 * D

        @pl.when(jnp.logical_not(is_last))
        def _():
            for p in range(0, CHUNK * D, PIECE):
                sz = min(PIECE, CHUNK * D - p)
                pltpu.sync_copy(x_hbm.at[pl.ds(base * D + p, sz)],
                                x_v.at[pl.ds(p, sz)])
            pltpu.sync_copy(ids_hbm.at[pl.ds(base, CHUNK)],
                            ids_v.at[pl.ds(0, CHUNK)])

        @pl.when(is_last)
        def _():
            for p in range(0, LASTC * D, PIECE):
                sz = min(PIECE, LASTC * D - p)
                pltpu.sync_copy(x_hbm.at[pl.ds((NW - 1) * CHUNK * D + p, sz)],
                                x_v.at[pl.ds(p, sz)])
            pltpu.sync_copy(ids_hbm.at[pl.ds((NW - 1) * CHUNK, LASTC)],
                            ids_v.at[pl.ds(0, LASTC)])

        zero16 = jnp.zeros((16,), jnp.float32)
        for s in range(G * NJ):
            acc_v[pl.ds(16 * s, 16)] = zero16

        ngroups = jnp.where(is_last, LGRP, NGRP)
        lane = lax.iota(jnp.int32, 16)
        prev0 = ids_v[pl.ds(0, 16)][0]
        init = (prev0, zero16) + tuple(zero16 for _ in range(NJ))

        def body(g, carry):
            prev, cntvec = carry[0], carry[1]
            accs = carry[2:]
            idvec = ids_v[pl.ds(16 * g, 16)]
            rowbase = g * (16 * D)
            for l in range(16):
                sid = idvec[l]
                flush = sid != prev

                @pl.when(flush)
                def _(prev=prev, accs=accs):
                    for j in range(NJ):
                        acc_v[pl.ds(prev * D + 16 * j, 16)] = accs[j]

                accs = tuple(
                    jnp.where(flush, 0.0, accs[j])
                    + x_v[pl.ds(rowbase + l * D + 16 * j, 16)]
                    for j in range(NJ)
                )
                cntvec = cntvec + jnp.where(lane == sid, 1.0, 0.0)
                prev = sid
            return (prev, cntvec) + accs

        final = lax.fori_loop(0, ngroups, body, init)
        prev, cntvec = final[0], final[1]
        accs = final[2:]
        for j in range(NJ):
            acc_v[pl.ds(prev * D + 16 * j, 16)] = accs[j]
        cnt_v[...] = cntvec

        pltpu.sync_copy(acc_v, sh_acc.at[pl.ds(wid * G * D, G * D)])
        pltpu.sync_copy(cnt_v, sh_cnt.at[pl.ds(wid * G, G)])
        pltpu.sync_copy(wct_hbm, wct_v)
        pltpu.sync_copy(bc_hbm, bc_v)
        plsc.subcore_barrier()

        for u in range(NW):
            pltpu.sync_copy(sh_acc.at[pl.ds(u * G * D + wid * D, D)],
                            x_v.at[pl.ds(u * D, D)])
        for j in range(NJ):
            s = zero16
            for u in range(NW):
                s = s + x_v[pl.ds(u * D + 16 * j, 16)]
            red_v[pl.ds(16 * j, 16)] = s

        for u in range(NW):
            pltpu.sync_copy(sh_cnt.at[pl.ds(u * G, G)], rcnt_v.at[pl.ds(u * G, G)])
        cnt_total = zero16
        for u in range(NW):
            cnt_total = cnt_total + rcnt_v[pl.ds(u * G, 16)]

        dn = lax.GatherDimensionNumbers(offset_dims=(),
                                        collapsed_slice_dims=(0,),
                                        start_index_map=(0,))

        def gat(v, idx):
            return lax.gather(v, idx[:, None], dn, (1,),
                              mode=lax.GatherScatterMode.PROMISE_IN_BOUNDS)

        cw_vec = gat(cnt_total, jnp.full((16,), 1, jnp.int32) * wid)
        invv = 1.0 / jnp.maximum(cw_vec, 1.0)

        outvec = bc_v[...]
        for c in range(7):
            acc = zero16
            for j in range(NJ):
                acc = acc + (red_v[pl.ds(16 * j, 16)]
                             * wct_v[pl.ds(c * D + 16 * j, 16)])
            for kk in (1, 2, 4, 8):
                acc = acc + gat(acc, jnp.bitwise_xor(lane, kk))
            outvec = outvec + jnp.where(lane == c, acc * invv, 0.0)
        out_v[...] = outvec
        pltpu.sync_copy(out_v, out_hbm.at[wid])

    return k(xf, ids, wct, bc16)


def kernel(x, edge_index, edge_attr, batch_size, W1, b1, W2, b2, Wc, bc):
    wct = jnp.zeros((8, D), jnp.float32).at[:7].set(Wc.T).reshape(-1)
    bc16 = jnp.zeros((16,), jnp.float32).at[:7].set(bc)
    padded = _sc_pool_classify(x.reshape(-1), batch_size, wct, bc16)
    return padded[:, :7]
